# baseline (speedup 1.0000x reference)
import jax
import jax.numpy as jnp
from jax.experimental import pallas as pl
from jax.experimental.pallas import tpu as pltpu

BATCH = 32
MAX_SEQ_LEN = 4096
KV_HEAD_DIM = 128
PREFILL_LEN = 2048

ZBLK = 512
N_ZERO_DMAS = (MAX_SEQ_LEN - PREFILL_LEN) // ZBLK
N_DMAS = 2 + 2 * N_ZERO_DMAS


def _body(kx_hbm, vx_hbm, k_out, v_out, zbuf, sems):
    zbuf[...] = jnp.zeros_like(zbuf)

    copies = []
    copies.append(
        pltpu.make_async_copy(
            kx_hbm, k_out.at[:, pl.ds(0, PREFILL_LEN), :], sems.at[0]
        )
    )
    copies.append(
        pltpu.make_async_copy(
            vx_hbm, v_out.at[:, pl.ds(0, PREFILL_LEN), :], sems.at[1]
        )
    )
    s = 2
    for out in (k_out, v_out):
        for i in range(N_ZERO_DMAS):
            copies.append(
                pltpu.make_async_copy(
                    zbuf,
                    out.at[:, pl.ds(PREFILL_LEN + i * ZBLK, ZBLK), :],
                    sems.at[s],
                )
            )
            s += 1
    for c in copies:
        c.start()
    for c in copies:
        c.wait()


def kernel(kx, vx):
    out_shape = jax.ShapeDtypeStruct((BATCH, MAX_SEQ_LEN, KV_HEAD_DIM), jnp.float32)
    any_spec = pl.BlockSpec(memory_space=pl.MemorySpace.ANY)
    return pl.pallas_call(
        _body,
        in_specs=[any_spec, any_spec],
        out_specs=[any_spec, any_spec],
        out_shape=[out_shape, out_shape],
        scratch_shapes=[
            pltpu.VMEM((BATCH, ZBLK, KV_HEAD_DIM), jnp.float32),
            pltpu.SemaphoreType.DMA((N_DMAS,)),
        ],
    )(kx, vx)


# speedup vs baseline: 24.2968x; 24.2968x over previous
import functools

import jax
import jax.numpy as jnp
from jax import lax
from jax.experimental import pallas as pl
from jax.experimental.pallas import tpu as pltpu
from jax.experimental.pallas import tpu_sc as plsc

BATCH = 32
MAX_SEQ_LEN = 4096
KV_HEAD_DIM = 128
PREFILL_LEN = 2048


SEQ_BLOCK = 256
N_BLOCKS = MAX_SEQ_LEN // SEQ_BLOCK
N_PREFILL_BLOCKS = PREFILL_LEN // SEQ_BLOCK


def _tc_body(kx_ref, k_out):
    j = pl.program_id(0)

    @pl.when(j < N_PREFILL_BLOCKS)
    def _copy():
        k_out[...] = kx_ref[...]

    @pl.when(j >= N_PREFILL_BLOCKS)
    def _zero():
        k_out[...] = jnp.zeros_like(k_out)


def _tc_cache(kx):
    in_spec = pl.BlockSpec(
        (BATCH, SEQ_BLOCK, KV_HEAD_DIM),
        lambda j: (0, jnp.minimum(j, N_PREFILL_BLOCKS - 1), 0),
    )
    out_spec = pl.BlockSpec(
        (BATCH, SEQ_BLOCK, KV_HEAD_DIM),
        lambda j: (0, j, 0),
    )
    return pl.pallas_call(
        _tc_body,
        grid=(N_BLOCKS,),
        in_specs=[in_spec],
        out_specs=out_spec,
        out_shape=jax.ShapeDtypeStruct((BATCH, MAX_SEQ_LEN, KV_HEAD_DIM), jnp.float32),
    )(kx)


NC = 2
NS = 16
CHUNK = 256
N_COPY = PREFILL_LEN // CHUNK
N_ZERO = (MAX_SEQ_LEN - PREFILL_LEN) // CHUNK


ZROWS = 64
N_ZERO_DMAS = (MAX_SEQ_LEN - PREFILL_LEN) // ZROWS


def _sc_body(vx_hbm, v_out, zb, buf0, buf1, sem_zs, sem_in, sem_out):
    c = lax.axis_index("c")
    s = lax.axis_index("s")
    wid = s * NC + c

    for r in range(ZROWS):
        for col in range(KV_HEAD_DIM // 16):
            zb[r, pl.ds(col * 16, 16)] = jnp.zeros((16,), jnp.float32)

    zeros_out = [
        pltpu.async_copy(
            zb, v_out.at[wid, pl.ds(PREFILL_LEN + i * ZROWS, ZROWS), :], sem_zs
        )
        for i in range(N_ZERO_DMAS)
    ]

    bufs = (buf0, buf1)
    in_d = [None] * N_COPY
    out_d = [None] * N_COPY
    in_d[0] = pltpu.async_copy(vx_hbm.at[wid, pl.ds(0, CHUNK), :], bufs[0], sem_in)
    in_d[1] = pltpu.async_copy(vx_hbm.at[wid, pl.ds(CHUNK, CHUNK), :], bufs[1], sem_in)
    for i in range(N_COPY):
        in_d[i].wait()
        out_d[i] = pltpu.async_copy(
            bufs[i % 2], v_out.at[wid, pl.ds(i * CHUNK, CHUNK), :], sem_out
        )
        nxt = i + 2
        if nxt < N_COPY:
            out_d[i].wait()
            in_d[nxt] = pltpu.async_copy(
                vx_hbm.at[wid, pl.ds(nxt * CHUNK, CHUNK), :], bufs[nxt % 2], sem_in
            )
    out_d[N_COPY - 2].wait()
    out_d[N_COPY - 1].wait()
    for d in zeros_out:
        d.wait()


def _sc_cache(vx):
    mesh = plsc.VectorSubcoreMesh(core_axis_name="c", subcore_axis_name="s")
    fn = functools.partial(
        pl.kernel,
        mesh=mesh,
        out_type=jax.ShapeDtypeStruct((BATCH, MAX_SEQ_LEN, KV_HEAD_DIM), jnp.float32),
        scratch_types=[
            pltpu.VMEM((ZROWS, KV_HEAD_DIM), jnp.float32),
            pltpu.VMEM((CHUNK, KV_HEAD_DIM), jnp.float32),
            pltpu.VMEM((CHUNK, KV_HEAD_DIM), jnp.float32),
            pltpu.SemaphoreType.DMA,
            pltpu.SemaphoreType.DMA,
            pltpu.SemaphoreType.DMA,
        ],
    )(_sc_body)
    return fn(vx)


def kernel(kx, vx):
    k_cache = _tc_cache(kx)
    v_cache = _sc_cache(vx)
    return (k_cache, v_cache)


# speedup vs baseline: 24.6392x; 1.0141x over previous
import functools

import jax
import jax.numpy as jnp
from jax import lax
from jax.experimental import pallas as pl
from jax.experimental.pallas import tpu as pltpu
from jax.experimental.pallas import tpu_sc as plsc

BATCH = 32
MAX_SEQ_LEN = 4096
KV_HEAD_DIM = 128
PREFILL_LEN = 2048


SEQ_BLOCK = 256
N_BLOCKS = MAX_SEQ_LEN // SEQ_BLOCK
N_PREFILL_BLOCKS = PREFILL_LEN // SEQ_BLOCK


def _tc_body(kx_ref, k_out):
    j = pl.program_id(0)

    @pl.when(j < N_PREFILL_BLOCKS)
    def _copy():
        k_out[...] = kx_ref[...]

    @pl.when(j >= N_PREFILL_BLOCKS)
    def _zero():
        k_out[...] = jnp.zeros_like(k_out)


def _tc_cache(kx):
    in_spec = pl.BlockSpec(
        (BATCH, SEQ_BLOCK, KV_HEAD_DIM),
        lambda j: (0, jnp.minimum(j, N_PREFILL_BLOCKS - 1), 0),
    )
    out_spec = pl.BlockSpec(
        (BATCH, SEQ_BLOCK, KV_HEAD_DIM),
        lambda j: (0, j, 0),
    )
    return pl.pallas_call(
        _tc_body,
        grid=(N_BLOCKS,),
        in_specs=[in_spec],
        out_specs=out_spec,
        out_shape=jax.ShapeDtypeStruct((BATCH, MAX_SEQ_LEN, KV_HEAD_DIM), jnp.float32),
    )(kx)


NC = 2
NS = 16
CHUNK = 256
N_COPY = PREFILL_LEN // CHUNK
N_ZERO = (MAX_SEQ_LEN - PREFILL_LEN) // CHUNK


ZROWS = 128
N_ZERO_DMAS = (MAX_SEQ_LEN - PREFILL_LEN) // ZROWS
NBUF = 3


def _sc_body(vx_hbm, v_out, zb, buf0, buf1, buf2, sem_zs, sem_in, sem_out):
    c = lax.axis_index("c")
    s = lax.axis_index("s")
    wid = s * NC + c

    for r in range(ZROWS):
        for col in range(KV_HEAD_DIM // 16):
            zb[r, pl.ds(col * 16, 16)] = jnp.zeros((16,), jnp.float32)

    zeros_out = [
        pltpu.async_copy(
            zb, v_out.at[wid, pl.ds(PREFILL_LEN + i * ZROWS, ZROWS), :], sem_zs
        )
        for i in range(N_ZERO_DMAS)
    ]

    bufs = (buf0, buf1, buf2)

    def src(i):
        return vx_hbm.at[wid, pl.ds(i * CHUNK, CHUNK), :]

    def dst(i):
        return v_out.at[wid, pl.ds(i * CHUNK, CHUNK), :]

    in_d = [None] * N_COPY
    out_d = [None] * N_COPY
    for b in range(min(NBUF, N_COPY)):
        in_d[b] = pltpu.async_copy(src(b), bufs[b], sem_in)
    for i in range(N_COPY):
        n = i + NBUF - 1
        if NBUF <= n < N_COPY:
            out_d[n - NBUF].wait()
            in_d[n] = pltpu.async_copy(src(n), bufs[n % NBUF], sem_in)
        in_d[i].wait()
        out_d[i] = pltpu.async_copy(bufs[i % NBUF], dst(i), sem_out)
    for i in range(max(0, N_COPY - NBUF), N_COPY):
        out_d[i].wait()
    for d in zeros_out:
        d.wait()


def _sc_cache(vx):
    mesh = plsc.VectorSubcoreMesh(core_axis_name="c", subcore_axis_name="s")
    fn = functools.partial(
        pl.kernel,
        mesh=mesh,
        out_type=jax.ShapeDtypeStruct((BATCH, MAX_SEQ_LEN, KV_HEAD_DIM), jnp.float32),
        scratch_types=[
            pltpu.VMEM((ZROWS, KV_HEAD_DIM), jnp.float32),
            pltpu.VMEM((CHUNK, KV_HEAD_DIM), jnp.float32),
            pltpu.VMEM((CHUNK, KV_HEAD_DIM), jnp.float32),
            pltpu.VMEM((CHUNK, KV_HEAD_DIM), jnp.float32),
            pltpu.SemaphoreType.DMA,
            pltpu.SemaphoreType.DMA,
            pltpu.SemaphoreType.DMA,
        ],
    )(_sc_body)
    return fn(vx)


def kernel(kx, vx):
    k_cache = _tc_cache(kx)
    v_cache = _sc_cache(vx)
    return (k_cache, v_cache)


# speedup vs baseline: 25.1822x; 1.0220x over previous
import functools

import jax
import jax.numpy as jnp
from jax import lax
from jax.experimental import pallas as pl
from jax.experimental.pallas import tpu as pltpu
from jax.experimental.pallas import tpu_sc as plsc

BATCH = 32
MAX_SEQ_LEN = 4096
KV_HEAD_DIM = 128
PREFILL_LEN = 2048


SEQ_BLOCK = 256
N_BLOCKS = MAX_SEQ_LEN // SEQ_BLOCK
N_PREFILL_BLOCKS = PREFILL_LEN // SEQ_BLOCK


def _tc_k_body(kx_ref, k_out):
    j = pl.program_id(0)

    @pl.when(j < N_PREFILL_BLOCKS)
    def _copy():
        k_out[...] = kx_ref[...]

    @pl.when(j >= N_PREFILL_BLOCKS)
    def _zero():
        k_out[...] = jnp.zeros_like(k_out)


def _tc_k(kx):
    in_spec = pl.BlockSpec(
        (BATCH, SEQ_BLOCK, KV_HEAD_DIM),
        lambda j: (0, jnp.minimum(j, N_PREFILL_BLOCKS - 1), 0),
    )
    out_spec = pl.BlockSpec(
        (BATCH, SEQ_BLOCK, KV_HEAD_DIM),
        lambda j: (0, j, 0),
    )
    return pl.pallas_call(
        _tc_k_body,
        grid=(N_BLOCKS,),
        in_specs=[in_spec],
        out_specs=out_spec,
        out_shape=jax.ShapeDtypeStruct((BATCH, MAX_SEQ_LEN, KV_HEAD_DIM), jnp.float32),
    )(kx)


NC = 2
NS = 16
ZROWS = 128
N_ZERO_DMAS = (MAX_SEQ_LEN - PREFILL_LEN) // ZROWS


def _sc_vzero_body(v_out, zb, sem_zs):
    c = lax.axis_index("c")
    s = lax.axis_index("s")
    wid = s * NC + c

    for r in range(ZROWS):
        for col in range(KV_HEAD_DIM // 16):
            zb[r, pl.ds(col * 16, 16)] = jnp.zeros((16,), jnp.float32)

    zeros_out = [
        pltpu.async_copy(
            zb, v_out.at[wid, pl.ds(PREFILL_LEN + i * ZROWS, ZROWS), :], sem_zs
        )
        for i in range(N_ZERO_DMAS)
    ]
    for d in zeros_out:
        d.wait()


def _sc_vzero():
    mesh = plsc.VectorSubcoreMesh(core_axis_name="c", subcore_axis_name="s")
    fn = functools.partial(
        pl.kernel,
        mesh=mesh,
        out_type=jax.ShapeDtypeStruct((BATCH, MAX_SEQ_LEN, KV_HEAD_DIM), jnp.float32),
        scratch_types=[
            pltpu.VMEM((ZROWS, KV_HEAD_DIM), jnp.float32),
            pltpu.SemaphoreType.DMA,
        ],
    )(_sc_vzero_body)
    return fn()


def _tc_vcopy_body(vpart_ref, vx_ref, v_out):
    del vpart_ref
    v_out[...] = vx_ref[...]


def _tc_vcopy(vpart, vx):
    blk = pl.BlockSpec(
        (BATCH, SEQ_BLOCK, KV_HEAD_DIM),
        lambda j: (0, j, 0),
    )
    return pl.pallas_call(
        _tc_vcopy_body,
        grid=(N_PREFILL_BLOCKS,),
        in_specs=[pl.BlockSpec(memory_space=pl.MemorySpace.ANY), blk],
        out_specs=blk,
        out_shape=jax.ShapeDtypeStruct((BATCH, MAX_SEQ_LEN, KV_HEAD_DIM), jnp.float32),
        input_output_aliases={0: 0},
    )(vpart, vx)


def kernel(kx, vx):
    vpart = _sc_vzero()
    k_cache = _tc_k(kx)
    v_cache = _tc_vcopy(vpart, vx)
    return (k_cache, v_cache)


# speedup vs baseline: 25.2742x; 1.0037x over previous
import functools

import jax
import jax.numpy as jnp
from jax import lax
from jax.experimental import pallas as pl
from jax.experimental.pallas import tpu as pltpu
from jax.experimental.pallas import tpu_sc as plsc

BATCH = 32
MAX_SEQ_LEN = 4096
KV_HEAD_DIM = 128
PREFILL_LEN = 2048


SEQ_BLOCK = 256
N_BLOCKS = MAX_SEQ_LEN // SEQ_BLOCK
N_PREFILL_BLOCKS = PREFILL_LEN // SEQ_BLOCK


def _tc_k_body(kx_ref, k_out):
    j = pl.program_id(0)

    @pl.when(j < N_PREFILL_BLOCKS)
    def _copy():
        k_out[...] = kx_ref[...]

    @pl.when(j >= N_PREFILL_BLOCKS)
    def _zero():
        k_out[...] = jnp.zeros_like(k_out)


def _tc_k(kx):
    in_spec = pl.BlockSpec(
        (BATCH, SEQ_BLOCK, KV_HEAD_DIM),
        lambda j: (0, jnp.minimum(j, N_PREFILL_BLOCKS - 1), 0),
    )
    out_spec = pl.BlockSpec(
        (BATCH, SEQ_BLOCK, KV_HEAD_DIM),
        lambda j: (0, j, 0),
    )
    return pl.pallas_call(
        _tc_k_body,
        grid=(N_BLOCKS,),
        in_specs=[in_spec],
        out_specs=out_spec,
        out_shape=jax.ShapeDtypeStruct((BATCH, MAX_SEQ_LEN, KV_HEAD_DIM), jnp.float32),
    )(kx)


NC = 2
NS = 16
ZROWS = 128
N_ZERO_DMAS = (MAX_SEQ_LEN - PREFILL_LEN) // ZROWS


def _sc_vzero_body(v_out, zb, sem_zs):
    c = lax.axis_index("c")
    s = lax.axis_index("s")
    wid = s * NC + c

    for r in range(ZROWS):
        for col in range(KV_HEAD_DIM // 16):
            zb[r, pl.ds(col * 16, 16)] = jnp.zeros((16,), jnp.float32)

    zeros_out = [
        pltpu.async_copy(
            zb, v_out.at[wid, pl.ds(PREFILL_LEN + i * ZROWS, ZROWS), :], sem_zs
        )
        for i in range(N_ZERO_DMAS)
    ]
    for d in zeros_out:
        d.wait()


def _sc_vzero():
    mesh = plsc.VectorSubcoreMesh(core_axis_name="c", subcore_axis_name="s")
    fn = functools.partial(
        pl.kernel,
        mesh=mesh,
        out_type=jax.ShapeDtypeStruct((BATCH, MAX_SEQ_LEN, KV_HEAD_DIM), jnp.float32),
        scratch_types=[
            pltpu.VMEM((ZROWS, KV_HEAD_DIM), jnp.float32),
            pltpu.SemaphoreType.DMA,
        ],
    )(_sc_vzero_body)
    return fn()


def _tc_vcopy_body(vpart_ref, vx_ref, k_ref, v_out):
    del vpart_ref, k_ref
    v_out[...] = vx_ref[...]


def _tc_vcopy(vpart, vx, k_cache):
    blk = pl.BlockSpec(
        (BATCH, SEQ_BLOCK, KV_HEAD_DIM),
        lambda j: (0, j, 0),
    )
    any_spec = pl.BlockSpec(memory_space=pl.MemorySpace.ANY)
    return pl.pallas_call(
        _tc_vcopy_body,
        grid=(N_PREFILL_BLOCKS,),
        in_specs=[any_spec, blk, any_spec],
        out_specs=blk,
        out_shape=jax.ShapeDtypeStruct((BATCH, MAX_SEQ_LEN, KV_HEAD_DIM), jnp.float32),
        input_output_aliases={0: 0},
    )(vpart, vx, k_cache)


def kernel(kx, vx):
    vpart = _sc_vzero()
    k_cache = _tc_k(kx)
    v_cache = _tc_vcopy(vpart, vx, k_cache)
    return (k_cache, v_cache)


# speedup vs baseline: 31.7215x; 1.2551x over previous
import jax
import jax.numpy as jnp
from jax.experimental import pallas as pl

BATCH = 32
MAX_SEQ_LEN = 4096
KV_HEAD_DIM = 128
PREFILL_LEN = 2048

SEQ_BLOCK = 128
N_BLOCKS = MAX_SEQ_LEN // SEQ_BLOCK
N_PREFILL_BLOCKS = PREFILL_LEN // SEQ_BLOCK


def _body(kx_ref, vx_ref, k_out, v_out):
    j = pl.program_id(0)

    @pl.when(j < N_PREFILL_BLOCKS)
    def _copy():
        k_out[...] = kx_ref[...]
        v_out[...] = vx_ref[...]

    @pl.when(j >= N_PREFILL_BLOCKS)
    def _zero():
        k_out[...] = jnp.zeros_like(k_out)
        v_out[...] = jnp.zeros_like(v_out)


def kernel(kx, vx):
    in_spec = pl.BlockSpec(
        (BATCH, SEQ_BLOCK, KV_HEAD_DIM),
        lambda j: (0, jnp.minimum(j, N_PREFILL_BLOCKS - 1), 0),
    )
    out_spec = pl.BlockSpec(
        (BATCH, SEQ_BLOCK, KV_HEAD_DIM),
        lambda j: (0, j, 0),
    )
    out_shape = jax.ShapeDtypeStruct((BATCH, MAX_SEQ_LEN, KV_HEAD_DIM), jnp.float32)
    return pl.pallas_call(
        _body,
        grid=(N_BLOCKS,),
        in_specs=[in_spec, in_spec],
        out_specs=[out_spec, out_spec],
        out_shape=[out_shape, out_shape],
    )(kx, vx)
